# baseline (speedup 1.0000x reference)
import jax
import jax.numpy as jnp
from jax import lax
from jax.experimental import pallas as pl
from jax.experimental.pallas import tpu as pltpu
from jax.experimental.pallas import tpu_sc as plsc

B, H, W, C = 4, 128, 128, 192
H2, W2 = 2 * H, 2 * W
N_IN = H * W * C
N_OUT = H2 * W2 * C

NC, NS, L = 2, 16, 16
WIN = 1_572_864
PASSES = N_OUT // WIN
TRASH = 8192
PER_TILE = N_IN // NS
CHUNK = 6144
N_CHUNKS = PER_TILE // CHUNK
ZCHUNK = 4096
TILE_WIN = WIN // NS
BATCHES_PER_CORE = B // NC
NSETS = 2
N_GROUPS = N_CHUNKS // NSETS


def _unpool_body(upd_hbm, mask_hbm, out_hbm,
                 idx_v0, idx_v1, val_v0, val_v1, zero_v, win_sh,
                 isem0, isem1, vsem0, vsem1, asem0, asem1):
    idx_v = (idx_v0, idx_v1)
    val_v = (val_v0, val_v1)
    isem = (isem0, isem1)
    vsem = (vsem0, vsem1)
    asem = (asem0, asem1)
    c = lax.axis_index("c")
    s_axis = lax.axis_index("s")

    def zfill(j, cv):
        zero_v[pl.ds(j * L, L)] = jnp.zeros((L,), jnp.float32)
        return cv

    lax.fori_loop(0, ZCHUNK // L, zfill, 0)

    def fire_in(i, s, in_base):
        start = in_base + i * CHUNK
        pltpu.async_copy(mask_hbm.at[pl.ds(start, CHUNK)], idx_v[s], isem[s])
        pltpu.async_copy(upd_hbm.at[pl.ds(start, CHUNK)], val_v[s], vsem[s])

    def wait_in(i, s, in_base):
        start = in_base + i * CHUNK
        pltpu.make_async_copy(mask_hbm.at[pl.ds(start, CHUNK)], idx_v[s],
                              isem[s]).wait()
        pltpu.make_async_copy(upd_hbm.at[pl.ds(start, CHUNK)], val_v[s],
                              vsem[s]).wait()

    def fire_add(s):
        pltpu.async_copy(val_v[s], win_sh.at[idx_v[s]], asem[s], add=True)

    def wait_add(s):
        pltpu.make_async_copy(val_v[s], win_sh.at[idx_v[s]], asem[s]).wait()

    def pass_body(bp, carry):
        bi = bp >> 3
        p = bp & (PASSES - 1)
        b = bi * NC + c
        lo = p * WIN
        in_base = b * N_IN + s_axis * PER_TILE

        def zcopy(z, cv):
            pltpu.sync_copy(
                zero_v,
                win_sh.at[pl.ds(s_axis * TILE_WIN + z * ZCHUNK, ZCHUNK)])
            return cv

        lax.fori_loop(0, TILE_WIN // ZCHUNK, zcopy, 0)
        plsc.subcore_barrier()

        fire_in(0, 0, in_base)

        def group_body(g, carry2):
            for s in range(NSETS):
                i = g * NSETS + s
                s2 = s ^ 1
                wait_in(i, s, in_base)

                @pl.when(i >= 1)
                def _():
                    wait_add(s2)

                @pl.when(i <= N_CHUNKS - 2)
                def _():
                    fire_in(i + 1, s2, in_base)

                @plsc.parallel_loop(0, CHUNK, L, unroll=4)
                def _(o, s=s):
                    iv = idx_v[s][pl.ds(o, L)]
                    u = iv - lo
                    inwin = plsc.bitcast(u, jnp.uint32) < jnp.uint32(WIN)
                    idx_v[s][pl.ds(o, L)] = jnp.where(
                        inwin, u, WIN + (iv & (TRASH - 1)))

                fire_add(s)
            return carry2

        lax.fori_loop(0, N_GROUPS, group_body, 0)
        wait_add((N_CHUNKS - 1) % NSETS)
        plsc.subcore_barrier()

        out_start = b * N_OUT + lo + s_axis * TILE_WIN
        pltpu.sync_copy(win_sh.at[pl.ds(s_axis * TILE_WIN, TILE_WIN)],
                        out_hbm.at[pl.ds(out_start, TILE_WIN)])
        return carry

    lax.fori_loop(0, BATCHES_PER_CORE * PASSES, pass_body, 0)


_unpool = pl.kernel(
    _unpool_body,
    out_type=jax.ShapeDtypeStruct((B * N_OUT,), jnp.float32),
    mesh=plsc.VectorSubcoreMesh(core_axis_name="c", subcore_axis_name="s",
                                num_cores=NC, num_subcores=NS),
    compiler_params=pltpu.CompilerParams(needs_layout_passes=False),
    scratch_types=(
        [pltpu.VMEM((CHUNK,), jnp.int32) for _ in range(NSETS)]
        + [pltpu.VMEM((CHUNK,), jnp.float32) for _ in range(NSETS)]
        + [pltpu.VMEM((ZCHUNK,), jnp.float32)]
        + [pltpu.VMEM_SHARED((WIN + TRASH,), jnp.float32)]
        + [pltpu.SemaphoreType.DMA] * (3 * NSETS)
    ),
)


@jax.jit
def kernel(updates, mask):
    upd = updates.reshape(-1)
    msk = mask.reshape(-1).astype(jnp.int32)
    out = _unpool(upd, msk)
    return out.reshape(B, H2, W2, C)


# speedup vs baseline: 1.0086x; 1.0086x over previous
import jax
import jax.numpy as jnp
from jax import lax
from jax.experimental import pallas as pl
from jax.experimental.pallas import tpu as pltpu
from jax.experimental.pallas import tpu_sc as plsc

B, H, W, C = 4, 128, 128, 192
H2, W2 = 2 * H, 2 * W
N_IN = H * W * C
N_OUT = H2 * W2 * C

NC, NS, L = 2, 16, 16
WIN = 1_572_864
PASSES = N_OUT // WIN
TRASH = 8192
PER_TILE = N_IN // NS
CHUNK = 6144
N_CHUNKS = PER_TILE // CHUNK
ZCHUNK = 4096
TILE_WIN = WIN // NS
BATCHES_PER_CORE = B // NC
NSETS = 2
N_GROUPS = N_CHUNKS // NSETS


def _unpool_body(upd_hbm, mask_hbm, out_hbm,
                 idx_v0, idx_v1, val_v0, val_v1, zero_v, win_sh,
                 isem0, isem1, vsem0, vsem1, asem0, asem1):
    idx_v = (idx_v0, idx_v1)
    val_v = (val_v0, val_v1)
    isem = (isem0, isem1)
    vsem = (vsem0, vsem1)
    asem = (asem0, asem1)
    c = lax.axis_index("c")
    s_axis = lax.axis_index("s")

    def zfill(j, cv):
        zero_v[pl.ds(j * L, L)] = jnp.zeros((L,), jnp.float32)
        return cv

    lax.fori_loop(0, ZCHUNK // L, zfill, 0)

    def fire_in(i, s, in_base):
        start = in_base + i * CHUNK
        pltpu.async_copy(mask_hbm.at[pl.ds(start, CHUNK)], idx_v[s], isem[s])
        pltpu.async_copy(upd_hbm.at[pl.ds(start, CHUNK)], val_v[s], vsem[s])

    def wait_in(i, s, in_base):
        start = in_base + i * CHUNK
        pltpu.make_async_copy(mask_hbm.at[pl.ds(start, CHUNK)], idx_v[s],
                              isem[s]).wait()
        pltpu.make_async_copy(upd_hbm.at[pl.ds(start, CHUNK)], val_v[s],
                              vsem[s]).wait()

    def fire_add(s):
        pltpu.async_copy(val_v[s], win_sh.at[idx_v[s]], asem[s], add=True)

    def wait_add(s):
        pltpu.make_async_copy(val_v[s], win_sh.at[idx_v[s]], asem[s]).wait()

    def pass_body(bp, carry):
        bi = bp >> 3
        p = bp & (PASSES - 1)
        b = bi * NC + c
        lo = p * WIN
        in_base = b * N_IN + s_axis * PER_TILE

        fire_in(0, 0, in_base)

        def zcopy(z, cv):
            pltpu.sync_copy(
                zero_v,
                win_sh.at[pl.ds(s_axis * TILE_WIN + z * ZCHUNK, ZCHUNK)])
            return cv

        lax.fori_loop(0, TILE_WIN // ZCHUNK, zcopy, 0)
        plsc.subcore_barrier()

        def group_body(g, carry2):
            for s in range(NSETS):
                i = g * NSETS + s
                s2 = s ^ 1
                wait_in(i, s, in_base)

                @pl.when(i >= 1)
                def _():
                    wait_add(s2)

                @pl.when(i <= N_CHUNKS - 2)
                def _():
                    fire_in(i + 1, s2, in_base)

                @plsc.parallel_loop(0, CHUNK, L, unroll=4)
                def _(o, s=s):
                    iv = idx_v[s][pl.ds(o, L)]
                    u = plsc.bitcast(iv - lo, jnp.uint32)
                    t = plsc.bitcast(WIN | (iv & (TRASH - 1)), jnp.uint32)
                    idx_v[s][pl.ds(o, L)] = plsc.bitcast(
                        jnp.minimum(u, t), jnp.int32)

                fire_add(s)
            return carry2

        lax.fori_loop(0, N_GROUPS, group_body, 0)
        wait_add((N_CHUNKS - 1) % NSETS)
        plsc.subcore_barrier()

        out_start = b * N_OUT + lo + s_axis * TILE_WIN
        pltpu.sync_copy(win_sh.at[pl.ds(s_axis * TILE_WIN, TILE_WIN)],
                        out_hbm.at[pl.ds(out_start, TILE_WIN)])
        return carry

    lax.fori_loop(0, BATCHES_PER_CORE * PASSES, pass_body, 0)


_unpool = pl.kernel(
    _unpool_body,
    out_type=jax.ShapeDtypeStruct((B * N_OUT,), jnp.float32),
    mesh=plsc.VectorSubcoreMesh(core_axis_name="c", subcore_axis_name="s",
                                num_cores=NC, num_subcores=NS),
    compiler_params=pltpu.CompilerParams(needs_layout_passes=False),
    scratch_types=(
        [pltpu.VMEM((CHUNK,), jnp.int32) for _ in range(NSETS)]
        + [pltpu.VMEM((CHUNK,), jnp.float32) for _ in range(NSETS)]
        + [pltpu.VMEM((ZCHUNK,), jnp.float32)]
        + [pltpu.VMEM_SHARED((WIN + TRASH,), jnp.float32)]
        + [pltpu.SemaphoreType.DMA] * (3 * NSETS)
    ),
)


@jax.jit
def kernel(updates, mask):
    upd = updates.reshape(-1)
    msk = mask.reshape(-1).astype(jnp.int32)
    out = _unpool(upd, msk)
    return out.reshape(B, H2, W2, C)


# speedup vs baseline: 1.1685x; 1.1585x over previous
import jax
import jax.numpy as jnp
from jax import lax
from jax.experimental import pallas as pl
from jax.experimental.pallas import tpu as pltpu
from jax.experimental.pallas import tpu_sc as plsc

B, H, W, C = 4, 128, 128, 192
H2, W2 = 2 * H, 2 * W
N_IN = H * W * C
N_OUT = H2 * W2 * C

NC, NS, L = 2, 16, 16
WIN = 1_572_864
PASSES = N_OUT // WIN
TRASH = 8192
PER_TILE = N_IN // NS
CHUNK = 4096
N_CHUNKS = PER_TILE // CHUNK
COLCAP = CHUNK // L
SEG = 512
CCAP = CHUNK + SEG
ZCHUNK = 4096
TILE_WIN = WIN // NS
BATCHES_PER_CORE = B // NC
NSETS = 2
N_GROUPS = N_CHUNKS // NSETS


def _unpool_body(upd_hbm, mask_hbm, out_hbm,
                 idx_v0, idx_v1, val_v0, val_v1, zero_v, cidx, cval, win_sh,
                 isem0, isem1, vsem0, vsem1):
    idx_v = (idx_v0, idx_v1)
    val_v = (val_v0, val_v1)
    isem = (isem0, isem1)
    vsem = (vsem0, vsem1)
    lane = lax.iota(jnp.int32, L)
    c = lax.axis_index("c")
    s_axis = lax.axis_index("s")

    def zfill(j, cv):
        zero_v[pl.ds(j * L, L)] = jnp.zeros((L,), jnp.float32)
        return cv

    lax.fori_loop(0, ZCHUNK // L, zfill, 0)

    def fire_in(i, s, in_base):
        start = in_base + i * CHUNK
        pltpu.async_copy(mask_hbm.at[pl.ds(start, CHUNK)], idx_v[s], isem[s])
        pltpu.async_copy(upd_hbm.at[pl.ds(start, CHUNK)], val_v[s], vsem[s])

    def wait_in(i, s, in_base):
        start = in_base + i * CHUNK
        pltpu.make_async_copy(mask_hbm.at[pl.ds(start, CHUNK)], idx_v[s],
                              isem[s]).wait()
        pltpu.make_async_copy(upd_hbm.at[pl.ds(start, CHUNK)], val_v[s],
                              vsem[s]).wait()

    def pass_body(bp, carry):
        bi = bp >> 3
        p = bp & (PASSES - 1)
        b = bi * NC + c
        lo = p * WIN
        in_base = b * N_IN + s_axis * PER_TILE

        fire_in(0, 0, in_base)

        def zcopy(z, cv):
            pltpu.sync_copy(
                zero_v,
                win_sh.at[pl.ds(s_axis * TILE_WIN + z * ZCHUNK, ZCHUNK)])
            return cv

        lax.fori_loop(0, TILE_WIN // ZCHUNK, zcopy, 0)
        plsc.subcore_barrier()

        def group_body(g, carry2):
            for s in range(NSETS):
                i = g * NSETS + s
                s2 = s ^ 1
                wait_in(i, s, in_base)

                @pl.when(i <= N_CHUNKS - 2)
                def _():
                    fire_in(i + 1, s2, in_base)

                def cbody(o, col, s=s):
                    iv = idx_v[s][pl.ds(o, L)]
                    u = iv - lo
                    m = plsc.bitcast(u, jnp.uint32) < jnp.uint32(WIN)
                    vv = val_v[s][pl.ds(o, L)]
                    pos = col * L + lane
                    plsc.store_scatter(cidx, [pos], u, mask=m)
                    plsc.store_scatter(cval, [pos], vv, mask=m)
                    return col + m.astype(jnp.int32)

                col = plsc.parallel_loop(
                    0, CHUNK, L, unroll=4,
                    carry=jnp.zeros((L,), jnp.int32))(cbody)

                mx = plsc.cummax(col)[L - 1]
                rows = ((mx + SEG // L - 1) // (SEG // L)) * (SEG // L)

                def pbody(r, cv):
                    pm = col <= r
                    pos = r * L + lane
                    plsc.store_scatter(cidx, [pos], lane * 1024 + (r & 1023),
                                       mask=pm)
                    plsc.store_scatter(cval, [pos],
                                       jnp.zeros((L,), jnp.float32), mask=pm)
                    return cv

                lax.fori_loop(0, rows, pbody, 0)

                def seg_body(k, cv):
                    pltpu.sync_copy(
                        cval.at[pl.ds(k * SEG, SEG)],
                        win_sh.at[cidx.at[pl.ds(k * SEG, SEG)]],
                        add=True)
                    return cv

                lax.fori_loop(0, rows * L // SEG, seg_body, 0)
            return carry2

        lax.fori_loop(0, N_GROUPS, group_body, 0)
        plsc.subcore_barrier()

        out_start = b * N_OUT + lo + s_axis * TILE_WIN
        pltpu.sync_copy(win_sh.at[pl.ds(s_axis * TILE_WIN, TILE_WIN)],
                        out_hbm.at[pl.ds(out_start, TILE_WIN)])
        return carry

    lax.fori_loop(0, BATCHES_PER_CORE * PASSES, pass_body, 0)


_unpool = pl.kernel(
    _unpool_body,
    out_type=jax.ShapeDtypeStruct((B * N_OUT,), jnp.float32),
    mesh=plsc.VectorSubcoreMesh(core_axis_name="c", subcore_axis_name="s",
                                num_cores=NC, num_subcores=NS),
    compiler_params=pltpu.CompilerParams(needs_layout_passes=False),
    scratch_types=(
        [pltpu.VMEM((CHUNK,), jnp.int32) for _ in range(NSETS)]
        + [pltpu.VMEM((CHUNK,), jnp.float32) for _ in range(NSETS)]
        + [pltpu.VMEM((ZCHUNK,), jnp.float32)]
        + [pltpu.VMEM((CCAP,), jnp.int32), pltpu.VMEM((CCAP,), jnp.float32)]
        + [pltpu.VMEM_SHARED((WIN + TRASH,), jnp.float32)]
        + [pltpu.SemaphoreType.DMA] * (2 * NSETS)
    ),
)


@jax.jit
def kernel(updates, mask):
    upd = updates.reshape(-1)
    msk = mask.reshape(-1).astype(jnp.int32)
    out = _unpool(upd, msk)
    return out.reshape(B, H2, W2, C)
